# baseline (speedup 1.0000x reference)
import math

import jax
import jax.numpy as jnp
from jax import lax
from jax.experimental import pallas as pl
from jax.experimental.pallas import tpu as pltpu
from jax.experimental.pallas import tpu_sc as plsc

N = 10000
E = 160000
A = 10
F = 16
S = 4
R = 8
O = 16

NC = 2
NS = 16
NW = NC * NS

CHUNK = 128
NCHUNK = 1280
E2 = NCHUNK * CHUNK
CPW = NCHUNK // NW
GROUP = 8
NG = CPW // GROUP

ROWS_PER_TILE = N // NS

SCALE_EDGE = 1.0 / math.sqrt(float(R * F * S * O))
SCALE_NODE = 1.0 / math.sqrt(float(O * A * O))


def _sc_mesh():
    return plsc.VectorSubcoreMesh(
        core_axis_name="c", subcore_axis_name="s", num_cores=NC, num_subcores=NS
    )


def _gather_body(nf_hbm, idx_hbm, out_hbm, idx_v, rows_v, gsem):
    c = lax.axis_index("c")
    s = lax.axis_index("s")
    wid = s * NC + c
    start = wid * CPW

    pltpu.sync_copy(idx_hbm.at[pl.ds(start, CPW)], idx_v)

    def grp(g, carry):
        for k in range(GROUP):
            pltpu.async_copy(
                nf_hbm.at[idx_v.at[g * GROUP + k]],
                rows_v.at[pl.ds((g * GROUP + k) * CHUNK, CHUNK)],
                gsem,
            )

        @pl.when(g >= 1)
        def _drain():
            pltpu.make_async_copy(
                nf_hbm.at[pl.ds(0, GROUP * CHUNK)],
                rows_v.at[pl.ds(0, GROUP * CHUNK)],
                gsem,
            ).wait()

        return carry

    lax.fori_loop(0, NG, grp, 0)
    pltpu.make_async_copy(
        nf_hbm.at[pl.ds(0, GROUP * CHUNK)],
        rows_v.at[pl.ds(0, GROUP * CHUNK)],
        gsem,
    ).wait()
    pltpu.sync_copy(rows_v, out_hbm.at[pl.ds(start * CHUNK, CPW * CHUNK)])


def _sc_gather(node_feats, sender_chunks):
    return pl.kernel(
        _gather_body,
        out_type=jax.ShapeDtypeStruct((E2, F), jnp.float32),
        mesh=_sc_mesh(),
        compiler_params=pltpu.CompilerParams(use_tc_tiling_on_sc=False),
        scratch_types=[
            pltpu.VMEM((CPW, CHUNK), jnp.int32),
            pltpu.VMEM((CPW * CHUNK, F), jnp.float32),
            pltpu.SemaphoreType.DMA,
        ],
    )(node_feats, sender_chunks)


def _scatter_body(mji_hbm, idx_hbm, out_hbm, idx_v, rows_v, zbuf, acc, sem):
    c = lax.axis_index("c")
    s = lax.axis_index("s")
    wid = s * NC + c
    start = wid * CPW

    def zstep(i, carry):
        zbuf[i] = jnp.zeros((O,), jnp.float32)
        return carry

    lax.fori_loop(0, ROWS_PER_TILE, zstep, 0)

    pltpu.sync_copy(idx_hbm.at[pl.ds(start, CPW)], idx_v)
    pltpu.async_copy(
        mji_hbm.at[pl.ds(start * CHUNK, CPW * CHUNK)], rows_v, sem
    ).wait()

    pltpu.sync_copy(zbuf, acc.at[pl.ds(s * ROWS_PER_TILE, ROWS_PER_TILE)])
    plsc.subcore_barrier()

    def step(j, carry):
        pltpu.sync_copy(
            rows_v.at[pl.ds(j * CHUNK, CHUNK)], acc.at[idx_v.at[j]], add=True
        )
        return carry

    lax.fori_loop(0, CPW, step, 0)
    plsc.subcore_barrier()

    pltpu.sync_copy(
        acc.at[pl.ds(s * ROWS_PER_TILE, ROWS_PER_TILE)],
        out_hbm.at[c, pl.ds(s * ROWS_PER_TILE, ROWS_PER_TILE)],
    )


def _sc_scatter(mji, recv_chunks):
    return pl.kernel(
        _scatter_body,
        out_type=jax.ShapeDtypeStruct((NC, N, O), jnp.float32),
        mesh=_sc_mesh(),
        compiler_params=pltpu.CompilerParams(use_tc_tiling_on_sc=False),
        scratch_types=[
            pltpu.VMEM((CPW, CHUNK), jnp.int32),
            pltpu.VMEM((CPW * CHUNK, O), jnp.float32),
            pltpu.VMEM((ROWS_PER_TILE, O), jnp.float32),
            pltpu.VMEM_SHARED((N, O), jnp.float32),
            pltpu.SemaphoreType.DMA,
        ],
    )(mji, recv_chunks)


EB = 2048


def _edge_body(xs_ref, ef_ref, ea_ref, wcat_ref, wlin1_ref, out_ref):
    xs = xs_ref[...]
    ef = ef_ref[...]
    ea = ea_ref[...]

    jj = lax.broadcasted_iota(jnp.int32, (R, R * F), 1)
    rr = lax.broadcasted_iota(jnp.int32, (R, R * F), 0)
    rep8 = (jj // F == rr).astype(jnp.float32)
    jj2 = lax.broadcasted_iota(jnp.int32, (F, R * F), 1)
    ff = lax.broadcasted_iota(jnp.int32, (F, R * F), 0)
    til16 = (jj2 % F == ff).astype(jnp.float32)

    jj3 = lax.broadcasted_iota(jnp.int32, (S, S * O), 1)
    ss = lax.broadcasted_iota(jnp.int32, (S, S * O), 0)
    tilb4 = (jj3 // O == ss).astype(jnp.float32)
    jj4 = lax.broadcasted_iota(jnp.int32, (S * O, O), 0)
    oo = lax.broadcasted_iota(jnp.int32, (S * O, O), 1)
    summ = (jj4 % O == oo).astype(jnp.float32)

    v = jnp.dot(ef, rep8, preferred_element_type=jnp.float32) * jnp.dot(
        xs, til16, preferred_element_type=jnp.float32
    )
    p = jnp.dot(v, wcat_ref[...], preferred_element_type=jnp.float32)
    q = p * jnp.dot(ea, tilb4, preferred_element_type=jnp.float32)
    wl1t = jnp.dot(summ, wlin1_ref[...], preferred_element_type=jnp.float32)
    out_ref[...] = jnp.dot(q, wl1t, preferred_element_type=jnp.float32) * SCALE_EDGE


def _tc_edge(xs, edge_attrs, edge_feats, wcat, w_lin1):
    grid = (E2 // EB,)
    return pl.pallas_call(
        _edge_body,
        grid=grid,
        in_specs=[
            pl.BlockSpec((EB, F), lambda i: (i, 0)),
            pl.BlockSpec((EB, R), lambda i: (i, 0)),
            pl.BlockSpec((EB, S), lambda i: (i, 0)),
            pl.BlockSpec((R * F, S * O), lambda i: (0, 0)),
            pl.BlockSpec((O, O), lambda i: (0, 0)),
        ],
        out_specs=pl.BlockSpec((EB, O), lambda i: (i, 0)),
        out_shape=jax.ShapeDtypeStruct((E2, O), jnp.float32),
    )(xs, edge_feats, edge_attrs, wcat, w_lin1)


NB = 2000


def _node_body(p_ref, na_ref, wsk_ref, wlin2_ref, out_ref):
    m = p_ref[0] + p_ref[1]
    na = na_ref[...]

    jj = lax.broadcasted_iota(jnp.int32, (O, O * A), 1)
    ff = lax.broadcasted_iota(jnp.int32, (O, O * A), 0)
    rep16 = (jj // A == ff).astype(jnp.float32)
    jj2 = lax.broadcasted_iota(jnp.int32, (A, O * A), 1)
    aa = lax.broadcasted_iota(jnp.int32, (A, O * A), 0)
    til10 = (jj2 % A == aa).astype(jnp.float32)

    v2 = jnp.dot(m, rep16, preferred_element_type=jnp.float32) * jnp.dot(
        na, til10, preferred_element_type=jnp.float32
    )
    x1 = jnp.dot(v2, wsk_ref[...], preferred_element_type=jnp.float32)
    x2 = jnp.dot(x1, wlin2_ref[...], preferred_element_type=jnp.float32)
    out_ref[...] = m + x2 * SCALE_NODE


def _tc_node(partials, node_attrs, wsk, w_lin2):
    grid = (N // NB,)
    return pl.pallas_call(
        _node_body,
        grid=grid,
        in_specs=[
            pl.BlockSpec((NC, NB, O), lambda i: (0, i, 0)),
            pl.BlockSpec((NB, A), lambda i: (i, 0)),
            pl.BlockSpec((O * A, O), lambda i: (0, 0)),
            pl.BlockSpec((O, O), lambda i: (0, 0)),
        ],
        out_specs=pl.BlockSpec((NB, O), lambda i: (i, 0)),
        out_shape=jax.ShapeDtypeStruct((N, O), jnp.float32),
    )(partials, node_attrs, wsk, w_lin2)


def kernel(node_attrs, node_feats, edge_attrs, edge_feats, edge_index,
           W_tpw, W_lin1, W_skip, W_lin2):
    pad = E2 - E
    sender_chunks = jnp.concatenate(
        [edge_index[0], jnp.zeros((pad,), jnp.int32)]
    ).reshape(NCHUNK, CHUNK)
    recv_chunks = jnp.concatenate(
        [edge_index[1], jnp.zeros((pad,), jnp.int32)]
    ).reshape(NCHUNK, CHUNK)
    ef_p = jnp.concatenate([edge_feats, jnp.zeros((pad, R), jnp.float32)])
    ea_p = jnp.concatenate([edge_attrs, jnp.zeros((pad, S), jnp.float32)])
    wcat = W_tpw.reshape(R * F, S * O)
    wsk = W_skip.reshape(O * A, O)

    xs = _sc_gather(node_feats, sender_chunks)
    mji = _tc_edge(xs, ea_p, ef_p, wcat, W_lin1)
    partials = _sc_scatter(mji, recv_chunks)
    out = _tc_node(partials, node_attrs, wsk, W_lin2)
    return out


# speedup vs baseline: 2.0106x; 2.0106x over previous
import math

import jax
import jax.numpy as jnp
from jax import lax
from jax.experimental import pallas as pl
from jax.experimental.pallas import tpu as pltpu
from jax.experimental.pallas import tpu_sc as plsc

N = 10000
E = 160000
A = 10
F = 16
S = 4
R = 8
O = 16

NC = 2
NS = 16
NW = NC * NS

CHUNK = 128
NCHUNK = 1280
E2 = NCHUNK * CHUNK
CPW = NCHUNK // NW
EPW = CPW * CHUNK
GROUP = 8
NG = CPW // GROUP
GROWS = GROUP * CHUNK

HALF = EPW // 2
CPH = CPW // 2

ROWS_PER_TILE = N // NS

SCALE_EDGE = 1.0 / math.sqrt(float(R * F * S * O))
SCALE_NODE = 1.0 / math.sqrt(float(O * A * O))


def _sc_mesh():
    return plsc.VectorSubcoreMesh(
        core_axis_name="c", subcore_axis_name="s", num_cores=NC, num_subcores=NS
    )


def _lane():
    return lax.iota(jnp.int32, 16)


def _gather_body(nf_hbm, idx_hbm, out_hbm, idx_v, rows_v, xst_v, gsem):
    c = lax.axis_index("c")
    s = lax.axis_index("s")
    wid = s * NC + c
    start = wid * CPW

    pltpu.sync_copy(idx_hbm.at[pl.ds(start, CPW)], idx_v)
    lane = _lane()

    def grp(g, carry):
        for k in range(GROUP):
            pltpu.async_copy(
                nf_hbm.at[idx_v.at[g * GROUP + k]],
                rows_v.at[pl.ds(k * CHUNK, CHUNK)],
                gsem,
            )
        pltpu.make_async_copy(
            nf_hbm.at[pl.ds(0, GROWS)], rows_v, gsem
        ).wait()

        def tstep(b, carry2):
            base = b * 16
            col0 = g * GROWS + base
            for f in range(F):
                vals = plsc.load_gather(
                    rows_v, [lane + base, jnp.full((16,), f, jnp.int32)]
                )
                xst_v[f, pl.ds(col0, 16)] = vals
            return carry2

        lax.fori_loop(0, GROWS // 16, tstep, 0)
        return carry

    lax.fori_loop(0, NG, grp, 0)

    for f in range(F):
        pltpu.sync_copy(xst_v.at[f], out_hbm.at[f, pl.ds(wid * EPW, EPW)])


def _sc_gather(node_feats, sender_chunks):
    return pl.kernel(
        _gather_body,
        out_type=jax.ShapeDtypeStruct((F, E2), jnp.float32),
        mesh=_sc_mesh(),
        compiler_params=pltpu.CompilerParams(use_tc_tiling_on_sc=False, needs_layout_passes=False),
        scratch_types=[
            pltpu.VMEM((CPW, CHUNK), jnp.int32),
            pltpu.VMEM((GROWS, F), jnp.float32),
            pltpu.VMEM((F, EPW), jnp.float32),
            pltpu.SemaphoreType.DMA,
        ],
    )(node_feats, sender_chunks)


def _scatter_body(mjit_hbm, idx_hbm, out_hbm, idx_v, mjit_v, rows_v, zbuf, acc, sem):
    c = lax.axis_index("c")
    s = lax.axis_index("s")
    wid = s * NC + c
    start = wid * CPW
    lane = _lane()

    def zstep(i, carry):
        zbuf[i] = jnp.zeros((O,), jnp.float32)
        return carry

    lax.fori_loop(0, ROWS_PER_TILE, zstep, 0)
    pltpu.sync_copy(idx_hbm.at[pl.ds(start, CPW)], idx_v)
    pltpu.sync_copy(zbuf, acc.at[pl.ds(s * ROWS_PER_TILE, ROWS_PER_TILE)])
    plsc.subcore_barrier()

    for p in range(2):
        for f in range(O):
            pltpu.sync_copy(
                mjit_hbm.at[f, pl.ds(wid * EPW + p * HALF, HALF)],
                mjit_v.at[f],
            )

        def tstep(b, carry2):
            base = b * 16
            for f in range(O):
                vals = mjit_v[f, pl.ds(base, 16)]
                plsc.store_scatter(
                    rows_v, [lane + base, jnp.full((16,), f, jnp.int32)], vals
                )
            return carry2

        lax.fori_loop(0, HALF // 16, tstep, 0)

        def step(j, carry2):
            pltpu.sync_copy(
                rows_v.at[pl.ds(j * CHUNK, CHUNK)],
                acc.at[idx_v.at[p * CPH + j]],
                add=True,
            )
            return carry2

        lax.fori_loop(0, CPH, step, 0)

    plsc.subcore_barrier()
    pltpu.sync_copy(
        acc.at[pl.ds(s * ROWS_PER_TILE, ROWS_PER_TILE)],
        out_hbm.at[c, pl.ds(s * ROWS_PER_TILE, ROWS_PER_TILE)],
    )


def _sc_scatter(mjit, recv_chunks):
    return pl.kernel(
        _scatter_body,
        out_type=jax.ShapeDtypeStruct((NC, N, O), jnp.float32),
        mesh=_sc_mesh(),
        compiler_params=pltpu.CompilerParams(use_tc_tiling_on_sc=False, needs_layout_passes=False),
        scratch_types=[
            pltpu.VMEM((CPW, CHUNK), jnp.int32),
            pltpu.VMEM((O, HALF), jnp.float32),
            pltpu.VMEM((HALF, O), jnp.float32),
            pltpu.VMEM((ROWS_PER_TILE, O), jnp.float32),
            pltpu.VMEM_SHARED((N, O), jnp.float32),
            pltpu.SemaphoreType.DMA,
        ],
    )(mjit, recv_chunks)


EB = 4096


def _edge_body(xst_ref, eat_ref, eft_ref, wcatt_ref, wlin1t_ref, out_ref):
    xst = xst_ref[...]
    eft = eft_ref[...]
    eat = eat_ref[...]

    ii = lax.broadcasted_iota(jnp.int32, (R * F, R), 0)
    rr = lax.broadcasted_iota(jnp.int32, (R * F, R), 1)
    rep8t = (ii // F == rr).astype(jnp.float32)
    ii2 = lax.broadcasted_iota(jnp.int32, (R * F, F), 0)
    ff = lax.broadcasted_iota(jnp.int32, (R * F, F), 1)
    til16t = (ii2 % F == ff).astype(jnp.float32)

    ii3 = lax.broadcasted_iota(jnp.int32, (S * O, S), 0)
    ss = lax.broadcasted_iota(jnp.int32, (S * O, S), 1)
    tilb4t = (ii3 // O == ss).astype(jnp.float32)
    oo = lax.broadcasted_iota(jnp.int32, (O, S * O), 0)
    jj4 = lax.broadcasted_iota(jnp.int32, (O, S * O), 1)
    summt = (jj4 % O == oo).astype(jnp.float32)

    vt = jnp.dot(rep8t, eft, preferred_element_type=jnp.float32) * jnp.dot(
        til16t, xst, preferred_element_type=jnp.float32
    )
    pt = jnp.dot(wcatt_ref[...], vt, preferred_element_type=jnp.float32)
    qt = pt * jnp.dot(tilb4t, eat, preferred_element_type=jnp.float32)
    wl1tt = jnp.dot(wlin1t_ref[...], summt, preferred_element_type=jnp.float32)
    out_ref[...] = jnp.dot(wl1tt, qt, preferred_element_type=jnp.float32) * SCALE_EDGE


def _tc_edge(xst, eat, eft, wcatt, wlin1t):
    grid = (E2 // EB,)
    return pl.pallas_call(
        _edge_body,
        grid=grid,
        in_specs=[
            pl.BlockSpec((F, EB), lambda i: (0, i)),
            pl.BlockSpec((S, EB), lambda i: (0, i)),
            pl.BlockSpec((R, EB), lambda i: (0, i)),
            pl.BlockSpec((S * O, R * F), lambda i: (0, 0)),
            pl.BlockSpec((O, O), lambda i: (0, 0)),
        ],
        out_specs=pl.BlockSpec((O, EB), lambda i: (0, i)),
        out_shape=jax.ShapeDtypeStruct((O, E2), jnp.float32),
    )(xst, eat, eft, wcatt, wlin1t)


NB = 2000


def _node_body(p_ref, na_ref, wsk_ref, wlin2_ref, out_ref):
    m = p_ref[0] + p_ref[1]
    na = na_ref[...]

    jj = lax.broadcasted_iota(jnp.int32, (O, O * A), 1)
    ff = lax.broadcasted_iota(jnp.int32, (O, O * A), 0)
    rep16 = (jj // A == ff).astype(jnp.float32)
    jj2 = lax.broadcasted_iota(jnp.int32, (A, O * A), 1)
    aa = lax.broadcasted_iota(jnp.int32, (A, O * A), 0)
    til10 = (jj2 % A == aa).astype(jnp.float32)

    v2 = jnp.dot(m, rep16, preferred_element_type=jnp.float32) * jnp.dot(
        na, til10, preferred_element_type=jnp.float32
    )
    x1 = jnp.dot(v2, wsk_ref[...], preferred_element_type=jnp.float32)
    x2 = jnp.dot(x1, wlin2_ref[...], preferred_element_type=jnp.float32)
    out_ref[...] = m + x2 * SCALE_NODE


def _tc_node(partials, node_attrs, wsk, w_lin2):
    grid = (N // NB,)
    return pl.pallas_call(
        _node_body,
        grid=grid,
        in_specs=[
            pl.BlockSpec((NC, NB, O), lambda i: (0, i, 0)),
            pl.BlockSpec((NB, A), lambda i: (i, 0)),
            pl.BlockSpec((O * A, O), lambda i: (0, 0)),
            pl.BlockSpec((O, O), lambda i: (0, 0)),
        ],
        out_specs=pl.BlockSpec((NB, O), lambda i: (i, 0)),
        out_shape=jax.ShapeDtypeStruct((N, O), jnp.float32),
    )(partials, node_attrs, wsk, w_lin2)


def kernel(node_attrs, node_feats, edge_attrs, edge_feats, edge_index,
           W_tpw, W_lin1, W_skip, W_lin2):
    pad = E2 - E
    sender_chunks = jnp.concatenate(
        [edge_index[0], jnp.zeros((pad,), jnp.int32)]
    ).reshape(NCHUNK, CHUNK)
    recv_chunks = jnp.concatenate(
        [edge_index[1], jnp.zeros((pad,), jnp.int32)]
    ).reshape(NCHUNK, CHUNK)
    eft = jnp.pad(edge_feats.T, ((0, 0), (0, pad)))
    eat = jnp.pad(edge_attrs.T, ((0, 0), (0, pad)))
    wcatt = W_tpw.reshape(R * F, S * O).T
    wlin1t = W_lin1.T
    wsk = W_skip.reshape(O * A, O)

    xst = _sc_gather(node_feats, sender_chunks)
    mjit = _tc_edge(xst, eat, eft, wcatt, wlin1t)
    partials = _sc_scatter(mjit, recv_chunks)
    out = _tc_node(partials, node_attrs, wsk, W_lin2)
    return out


# speedup vs baseline: 2.4608x; 1.2239x over previous
import math

import jax
import jax.numpy as jnp
from jax import lax
from jax.experimental import pallas as pl
from jax.experimental.pallas import tpu as pltpu
from jax.experimental.pallas import tpu_sc as plsc

N = 10000
E = 160000
A = 10
F = 16
S = 4
R = 8
O = 16

NC = 2
NS = 16
NW = NC * NS

CHUNK = 128
NCR = E // CHUNK
NCHUNK = 1280
E2 = NCHUNK * CHUNK
CPW = NCHUNK // NW
EPW = CPW * CHUNK
GROUP = 8
NG = CPW // GROUP
GROWS = GROUP * CHUNK

HALF = EPW // 2
CPH = CPW // 2

ROWS_PER_TILE = N // NS

SCALE_EDGE = 1.0 / math.sqrt(float(R * F * S * O))
SCALE_NODE = 1.0 / math.sqrt(float(O * A * O))


def _sc_mesh():
    return plsc.VectorSubcoreMesh(
        core_axis_name="c", subcore_axis_name="s", num_cores=NC, num_subcores=NS
    )


def _lane():
    return lax.iota(jnp.int32, 16)


def _gather_body(nf_hbm, idx_hbm, out_hbm, idx_v, rows_a, rows_b, xst_v,
                 sem_a, sem_b, wsem):
    c = lax.axis_index("c")
    s = lax.axis_index("s")
    wid = s * NC + c
    start = wid * CPW
    start_eff = jnp.minimum(start, NCR - CPW)
    lane = _lane()

    pltpu.sync_copy(idx_hbm.at[pl.ds(start_eff, CPW)], idx_v)

    def fire(g, rows_v, sem):
        for k in range(GROUP):
            row = jnp.minimum(start + g * GROUP + k, NCR - 1) - start_eff
            pltpu.async_copy(
                nf_hbm.at[idx_v.at[row]],
                rows_v.at[pl.ds(k * CHUNK, CHUNK)],
                sem,
            )

    def drain(rows_v, sem):
        pltpu.make_async_copy(nf_hbm.at[pl.ds(0, GROWS)], rows_v, sem).wait()

    def transpose(g, rows_v):
        def tstep(b, carry):
            base = b * 16
            col0 = g * GROWS + base
            for f in range(F):
                vals = plsc.load_gather(
                    rows_v, [lane + base, jnp.full((16,), f, jnp.int32)]
                )
                xst_v[f, pl.ds(col0, 16)] = vals
            return carry

        lax.fori_loop(0, GROWS // 16, tstep, 0)

    fire(0, rows_a, sem_a)

    def body(i, carry):
        g = 2 * i
        fire(g + 1, rows_b, sem_b)
        drain(rows_a, sem_a)
        transpose(g, rows_a)
        fire(g + 2, rows_a, sem_a)
        drain(rows_b, sem_b)
        transpose(g + 1, rows_b)
        return carry

    lax.fori_loop(0, (NG - 1) // 2, body, 0)
    drain(rows_a, sem_a)
    transpose(NG - 1, rows_a)

    pltpu.sync_copy(xst_v, out_hbm.at[:, pl.ds(wid * EPW, EPW)])


def _sc_gather(node_feats, sender_chunks):
    return pl.kernel(
        _gather_body,
        out_type=jax.ShapeDtypeStruct((F, E2), jnp.float32),
        mesh=_sc_mesh(),
        compiler_params=pltpu.CompilerParams(
            use_tc_tiling_on_sc=False, needs_layout_passes=False
        ),
        scratch_types=[
            pltpu.VMEM((CPW, CHUNK), jnp.int32),
            pltpu.VMEM((GROWS, F), jnp.float32),
            pltpu.VMEM((GROWS, F), jnp.float32),
            pltpu.VMEM((F, EPW), jnp.float32),
            pltpu.SemaphoreType.DMA,
            pltpu.SemaphoreType.DMA,
            pltpu.SemaphoreType.DMA,
        ],
    )(node_feats, sender_chunks)


def _scatter_body(mjit_hbm, idx_hbm, out_hbm, idx_v, mjit_v, rows_v, zbuf,
                  acc, sem):
    c = lax.axis_index("c")
    s = lax.axis_index("s")
    wid = s * NC + c
    start = wid * CPW
    start_eff = jnp.minimum(start, NCR - CPW)
    lane = _lane()

    def zstep(i, carry):
        zbuf[i] = jnp.zeros((O,), jnp.float32)
        return carry

    lax.fori_loop(0, ROWS_PER_TILE, zstep, 0)
    pltpu.sync_copy(idx_hbm.at[pl.ds(start_eff, CPW)], idx_v)
    pltpu.sync_copy(zbuf, acc.at[pl.ds(s * ROWS_PER_TILE, ROWS_PER_TILE)])
    plsc.subcore_barrier()

    for p in range(2):
        pltpu.sync_copy(
            mjit_hbm.at[:, pl.ds(wid * EPW + p * HALF, HALF)], mjit_v
        )

        def tstep(b, carry):
            base = b * 16
            for f in range(O):
                vals = mjit_v[f, pl.ds(base, 16)]
                plsc.store_scatter(
                    rows_v, [lane + base, jnp.full((16,), f, jnp.int32)], vals
                )
            return carry

        lax.fori_loop(0, HALF // 16, tstep, 0)

        def step(j, carry):
            row = jnp.minimum(start + p * CPH + j, NCR - 1) - start_eff
            pltpu.async_copy(
                rows_v.at[pl.ds(j * CHUNK, CHUNK)],
                acc.at[idx_v.at[row]],
                sem,
                add=True,
            )

            @pl.when(j >= 1)
            def _():
                pltpu.make_async_copy(
                    rows_v.at[pl.ds(0, CHUNK)],
                    acc.at[pl.ds(0, CHUNK)],
                    sem,
                ).wait()

            return carry

        lax.fori_loop(0, CPH, step, 0)
        pltpu.make_async_copy(
            rows_v.at[pl.ds(0, CHUNK)], acc.at[pl.ds(0, CHUNK)], sem
        ).wait()

    plsc.subcore_barrier()
    pltpu.sync_copy(
        acc.at[pl.ds(s * ROWS_PER_TILE, ROWS_PER_TILE)],
        out_hbm.at[c, pl.ds(s * ROWS_PER_TILE, ROWS_PER_TILE)],
    )


def _sc_scatter(mjit, recv_chunks):
    return pl.kernel(
        _scatter_body,
        out_type=jax.ShapeDtypeStruct((NC, N, O), jnp.float32),
        mesh=_sc_mesh(),
        compiler_params=pltpu.CompilerParams(
            use_tc_tiling_on_sc=False, needs_layout_passes=False
        ),
        scratch_types=[
            pltpu.VMEM((CPW, CHUNK), jnp.int32),
            pltpu.VMEM((O, HALF), jnp.float32),
            pltpu.VMEM((HALF, O), jnp.float32),
            pltpu.VMEM((ROWS_PER_TILE, O), jnp.float32),
            pltpu.VMEM_SHARED((N, O), jnp.float32),
            pltpu.SemaphoreType.DMA,
        ],
    )(mjit, recv_chunks)


EB = 4096


def _edge_body(xst_ref, eat_ref, eft_ref, wcatt_ref, wlin1t_ref, out_ref):
    xst = xst_ref[...]
    eft = eft_ref[...]
    eat = eat_ref[...]

    ii = lax.broadcasted_iota(jnp.int32, (R * F, R), 0)
    rr = lax.broadcasted_iota(jnp.int32, (R * F, R), 1)
    rep8t = (ii // F == rr).astype(jnp.float32)
    ii2 = lax.broadcasted_iota(jnp.int32, (R * F, F), 0)
    ff = lax.broadcasted_iota(jnp.int32, (R * F, F), 1)
    til16t = (ii2 % F == ff).astype(jnp.float32)

    ii3 = lax.broadcasted_iota(jnp.int32, (S * O, S), 0)
    ss = lax.broadcasted_iota(jnp.int32, (S * O, S), 1)
    tilb4t = (ii3 // O == ss).astype(jnp.float32)
    oo = lax.broadcasted_iota(jnp.int32, (O, S * O), 0)
    jj4 = lax.broadcasted_iota(jnp.int32, (O, S * O), 1)
    summt = (jj4 % O == oo).astype(jnp.float32)

    vt = jnp.dot(rep8t, eft, preferred_element_type=jnp.float32) * jnp.dot(
        til16t, xst, preferred_element_type=jnp.float32
    )
    pt = jnp.dot(wcatt_ref[...], vt, preferred_element_type=jnp.float32)
    qt = pt * jnp.dot(tilb4t, eat, preferred_element_type=jnp.float32)
    wl1tt = jnp.dot(wlin1t_ref[...], summt, preferred_element_type=jnp.float32)
    out_ref[...] = jnp.dot(wl1tt, qt, preferred_element_type=jnp.float32) * SCALE_EDGE


def _tc_edge(xst, eat, eft, wcatt, wlin1t):
    grid = (E2 // EB,)
    return pl.pallas_call(
        _edge_body,
        grid=grid,
        in_specs=[
            pl.BlockSpec((F, EB), lambda i: (0, i)),
            pl.BlockSpec((S, EB), lambda i: (0, i)),
            pl.BlockSpec((R, EB), lambda i: (0, i)),
            pl.BlockSpec((S * O, R * F), lambda i: (0, 0)),
            pl.BlockSpec((O, O), lambda i: (0, 0)),
        ],
        out_specs=pl.BlockSpec((O, EB), lambda i: (0, i)),
        out_shape=jax.ShapeDtypeStruct((O, E2), jnp.float32),
    )(xst, eat, eft, wcatt, wlin1t)


NB = 2000


def _node_body(p_ref, na_ref, wsk_ref, wlin2_ref, out_ref):
    m = p_ref[0] + p_ref[1]
    na = na_ref[...]

    jj = lax.broadcasted_iota(jnp.int32, (O, O * A), 1)
    ff = lax.broadcasted_iota(jnp.int32, (O, O * A), 0)
    rep16 = (jj // A == ff).astype(jnp.float32)
    jj2 = lax.broadcasted_iota(jnp.int32, (A, O * A), 1)
    aa = lax.broadcasted_iota(jnp.int32, (A, O * A), 0)
    til10 = (jj2 % A == aa).astype(jnp.float32)

    v2 = jnp.dot(m, rep16, preferred_element_type=jnp.float32) * jnp.dot(
        na, til10, preferred_element_type=jnp.float32
    )
    x1 = jnp.dot(v2, wsk_ref[...], preferred_element_type=jnp.float32)
    x2 = jnp.dot(x1, wlin2_ref[...], preferred_element_type=jnp.float32)
    out_ref[...] = m + x2 * SCALE_NODE


def _tc_node(partials, node_attrs, wsk, w_lin2):
    grid = (N // NB,)
    return pl.pallas_call(
        _node_body,
        grid=grid,
        in_specs=[
            pl.BlockSpec((NC, NB, O), lambda i: (0, i, 0)),
            pl.BlockSpec((NB, A), lambda i: (i, 0)),
            pl.BlockSpec((O * A, O), lambda i: (0, 0)),
            pl.BlockSpec((O, O), lambda i: (0, 0)),
        ],
        out_specs=pl.BlockSpec((NB, O), lambda i: (i, 0)),
        out_shape=jax.ShapeDtypeStruct((N, O), jnp.float32),
    )(partials, node_attrs, wsk, w_lin2)


def kernel(node_attrs, node_feats, edge_attrs, edge_feats, edge_index,
           W_tpw, W_lin1, W_skip, W_lin2):
    pad = E2 - E
    sender_chunks = edge_index[0].reshape(NCR, CHUNK)
    recv_chunks = edge_index[1].reshape(NCR, CHUNK)
    eft = jnp.pad(edge_feats.T, ((0, 0), (0, pad)))
    eat = jnp.pad(edge_attrs.T, ((0, 0), (0, pad)))
    wcatt = W_tpw.reshape(R * F, S * O).T
    wlin1t = W_lin1.T
    wsk = W_skip.reshape(O * A, O)

    xst = _sc_gather(node_feats, sender_chunks)
    mjit = _tc_edge(xst, eat, eft, wcatt, wlin1t)
    partials = _sc_scatter(mjit, recv_chunks)
    out = _tc_node(partials, node_attrs, wsk, W_lin2)
    return out


# speedup vs baseline: 2.7640x; 1.1232x over previous
import math

import jax
import jax.numpy as jnp
from jax import lax
from jax.experimental import pallas as pl
from jax.experimental.pallas import tpu as pltpu
from jax.experimental.pallas import tpu_sc as plsc

N = 10000
E = 160000
A = 10
F = 16
S = 4
R = 8
O = 16

NC = 2
NS = 16
NW = NC * NS

CHUNK = 128
NCR = E // CHUNK
NCHUNK = 1280
E2 = NCHUNK * CHUNK
CPW = NCHUNK // NW
EPW = CPW * CHUNK
GROUP = 8
NG = CPW // GROUP
GROWS = GROUP * CHUNK

HALF = EPW // 2
CPH = CPW // 2

ROWS_PER_TILE = N // NS

SCALE_EDGE = 1.0 / math.sqrt(float(R * F * S * O))
SCALE_NODE = 1.0 / math.sqrt(float(O * A * O))


def _sc_mesh():
    return plsc.VectorSubcoreMesh(
        core_axis_name="c", subcore_axis_name="s", num_cores=NC, num_subcores=NS
    )


def _lane():
    return lax.iota(jnp.int32, 16)


def _gather_body(nf_hbm, idx_hbm, out_hbm, idx_v, rows_a, rows_b, xst_v,
                 sem_a, sem_b, wsem):
    c = lax.axis_index("c")
    s = lax.axis_index("s")
    wid = s * NC + c
    start = wid * CPW
    start_eff = jnp.minimum(start, NCR - CPW)
    lane = _lane()

    pltpu.sync_copy(idx_hbm.at[pl.ds(start_eff, CPW)], idx_v)

    def fire(g, rows_v, sem):
        for k in range(GROUP):
            row = jnp.minimum(start + g * GROUP + k, NCR - 1) - start_eff
            pltpu.async_copy(
                nf_hbm.at[idx_v.at[row]],
                rows_v.at[pl.ds(k * CHUNK, CHUNK)],
                sem,
            )

    def drain(rows_v, sem):
        pltpu.make_async_copy(nf_hbm.at[pl.ds(0, GROWS)], rows_v, sem).wait()

    def transpose(g, rows_v):
        def tstep(b, carry):
            base = b * 16
            ct = g * (GROWS // 128) + b // 8
            cc = (b % 8) * 16
            for f in range(F):
                vals = plsc.load_gather(
                    rows_v, [lane + base, jnp.full((16,), f, jnp.int32)]
                )
                xst_v[f // 8, ct, f % 8, pl.ds(cc, 16)] = vals
            return carry

        lax.fori_loop(0, GROWS // 16, tstep, 0)

    fire(0, rows_a, sem_a)

    def body(i, carry):
        g = 2 * i
        fire(g + 1, rows_b, sem_b)
        drain(rows_a, sem_a)
        transpose(g, rows_a)
        fire(g + 2, rows_a, sem_a)
        drain(rows_b, sem_b)
        transpose(g + 1, rows_b)
        return carry

    lax.fori_loop(0, (NG - 1) // 2, body, 0)
    drain(rows_a, sem_a)
    transpose(NG - 1, rows_a)

    for rt in range(2):
        pltpu.sync_copy(
            xst_v.at[rt], out_hbm.at[rt, pl.ds(wid * CPW, CPW)]
        )


def _sc_gather(node_feats, sender_chunks):
    return pl.kernel(
        _gather_body,
        out_type=jax.ShapeDtypeStruct((2, NCHUNK, 8, 128), jnp.float32),
        mesh=_sc_mesh(),
        compiler_params=pltpu.CompilerParams(
            use_tc_tiling_on_sc=False, needs_layout_passes=False
        ),
        scratch_types=[
            pltpu.VMEM((CPW, CHUNK), jnp.int32),
            pltpu.VMEM((GROWS, F), jnp.float32),
            pltpu.VMEM((GROWS, F), jnp.float32),
            pltpu.VMEM((2, CPW, 8, 128), jnp.float32),
            pltpu.SemaphoreType.DMA,
            pltpu.SemaphoreType.DMA,
            pltpu.SemaphoreType.DMA,
        ],
    )(node_feats, sender_chunks)


def _scatter_body(mjit_hbm, idx_hbm, out_hbm, idx_v, mjit_v, rows_v, zbuf,
                  acc, sem):
    c = lax.axis_index("c")
    s = lax.axis_index("s")
    wid = s * NC + c
    start = wid * CPW
    start_eff = jnp.minimum(start, NCR - CPW)
    lane = _lane()

    def zstep(i, carry):
        zbuf[i] = jnp.zeros((O,), jnp.float32)
        return carry

    lax.fori_loop(0, ROWS_PER_TILE, zstep, 0)
    pltpu.sync_copy(idx_hbm.at[pl.ds(start_eff, CPW)], idx_v)
    pltpu.sync_copy(zbuf, acc.at[pl.ds(s * ROWS_PER_TILE, ROWS_PER_TILE)])
    plsc.subcore_barrier()

    for p in range(2):
        for rt in range(2):
            pltpu.sync_copy(
                mjit_hbm.at[rt, pl.ds(wid * CPW + p * CPH, CPH)],
                mjit_v.at[rt],
            )

        def tstep(b, carry):
            base = b * 16
            ct = b // 8
            cc = (b % 8) * 16
            for f in range(O):
                vals = mjit_v[f // 8, ct, f % 8, pl.ds(cc, 16)]
                plsc.store_scatter(
                    rows_v, [lane + base, jnp.full((16,), f, jnp.int32)], vals
                )
            return carry

        lax.fori_loop(0, HALF // 16, tstep, 0)

        def step(j, carry):
            row = jnp.minimum(start + p * CPH + j, NCR - 1) - start_eff
            pltpu.async_copy(
                rows_v.at[pl.ds(j * CHUNK, CHUNK)],
                acc.at[idx_v.at[row]],
                sem,
                add=True,
            )

            @pl.when(j >= 1)
            def _():
                pltpu.make_async_copy(
                    rows_v.at[pl.ds(0, CHUNK)],
                    acc.at[pl.ds(0, CHUNK)],
                    sem,
                ).wait()

            return carry

        lax.fori_loop(0, CPH, step, 0)
        pltpu.make_async_copy(
            rows_v.at[pl.ds(0, CHUNK)], acc.at[pl.ds(0, CHUNK)], sem
        ).wait()

    plsc.subcore_barrier()
    pltpu.sync_copy(
        acc.at[pl.ds(s * ROWS_PER_TILE, ROWS_PER_TILE)],
        out_hbm.at[c, pl.ds(s * ROWS_PER_TILE, ROWS_PER_TILE)],
    )


def _sc_scatter(mjit, recv_chunks):
    return pl.kernel(
        _scatter_body,
        out_type=jax.ShapeDtypeStruct((NC, N, O), jnp.float32),
        mesh=_sc_mesh(),
        compiler_params=pltpu.CompilerParams(
            use_tc_tiling_on_sc=False, needs_layout_passes=False
        ),
        scratch_types=[
            pltpu.VMEM((CPW, CHUNK), jnp.int32),
            pltpu.VMEM((2, CPH, 8, 128), jnp.float32),
            pltpu.VMEM((HALF, O), jnp.float32),
            pltpu.VMEM((ROWS_PER_TILE, O), jnp.float32),
            pltpu.VMEM_SHARED((N, O), jnp.float32),
            pltpu.SemaphoreType.DMA,
        ],
    )(mjit, recv_chunks)


EB = 4096


def _edge_body(xst_ref, eat_ref, eft_ref, wcatt_ref, wlin1t_ref, out_ref):
    xst = xst_ref[...].transpose(0, 2, 1, 3).reshape(F, EB)
    eft = eft_ref[...]
    eat = eat_ref[...]

    ii = lax.broadcasted_iota(jnp.int32, (R * F, R), 0)
    rr = lax.broadcasted_iota(jnp.int32, (R * F, R), 1)
    rep8t = (ii // F == rr).astype(jnp.float32)
    ii2 = lax.broadcasted_iota(jnp.int32, (R * F, F), 0)
    ff = lax.broadcasted_iota(jnp.int32, (R * F, F), 1)
    til16t = (ii2 % F == ff).astype(jnp.float32)

    ii3 = lax.broadcasted_iota(jnp.int32, (S * O, S), 0)
    ss = lax.broadcasted_iota(jnp.int32, (S * O, S), 1)
    tilb4t = (ii3 // O == ss).astype(jnp.float32)
    oo = lax.broadcasted_iota(jnp.int32, (O, S * O), 0)
    jj4 = lax.broadcasted_iota(jnp.int32, (O, S * O), 1)
    summt = (jj4 % O == oo).astype(jnp.float32)

    vt = jnp.dot(rep8t, eft, preferred_element_type=jnp.float32) * jnp.dot(
        til16t, xst, preferred_element_type=jnp.float32
    )
    pt = jnp.dot(wcatt_ref[...], vt, preferred_element_type=jnp.float32)
    qt = pt * jnp.dot(tilb4t, eat, preferred_element_type=jnp.float32)
    wl1tt = jnp.dot(wlin1t_ref[...], summt, preferred_element_type=jnp.float32)
    mjit = jnp.dot(wl1tt, qt, preferred_element_type=jnp.float32) * SCALE_EDGE
    out_ref[...] = mjit.reshape(2, 8, EB // 128, 128).transpose(0, 2, 1, 3)


def _tc_edge(xst, eat, eft, wcatt, wlin1t):
    grid = (E2 // EB,)
    return pl.pallas_call(
        _edge_body,
        grid=grid,
        in_specs=[
            pl.BlockSpec((2, EB // 128, 8, 128), lambda i: (0, i, 0, 0)),
            pl.BlockSpec((S, EB), lambda i: (0, i)),
            pl.BlockSpec((R, EB), lambda i: (0, i)),
            pl.BlockSpec((S * O, R * F), lambda i: (0, 0)),
            pl.BlockSpec((O, O), lambda i: (0, 0)),
        ],
        out_specs=pl.BlockSpec((2, EB // 128, 8, 128), lambda i: (0, i, 0, 0)),
        out_shape=jax.ShapeDtypeStruct((2, NCHUNK, 8, 128), jnp.float32),
    )(xst, eat, eft, wcatt, wlin1t)


NB = 2000


def _node_body(p_ref, na_ref, wsk_ref, wlin2_ref, out_ref):
    m = p_ref[0] + p_ref[1]
    na = na_ref[...]

    jj = lax.broadcasted_iota(jnp.int32, (O, O * A), 1)
    ff = lax.broadcasted_iota(jnp.int32, (O, O * A), 0)
    rep16 = (jj // A == ff).astype(jnp.float32)
    jj2 = lax.broadcasted_iota(jnp.int32, (A, O * A), 1)
    aa = lax.broadcasted_iota(jnp.int32, (A, O * A), 0)
    til10 = (jj2 % A == aa).astype(jnp.float32)

    v2 = jnp.dot(m, rep16, preferred_element_type=jnp.float32) * jnp.dot(
        na, til10, preferred_element_type=jnp.float32
    )
    x1 = jnp.dot(v2, wsk_ref[...], preferred_element_type=jnp.float32)
    x2 = jnp.dot(x1, wlin2_ref[...], preferred_element_type=jnp.float32)
    out_ref[...] = m + x2 * SCALE_NODE


def _tc_node(partials, node_attrs, wsk, w_lin2):
    grid = (N // NB,)
    return pl.pallas_call(
        _node_body,
        grid=grid,
        in_specs=[
            pl.BlockSpec((NC, NB, O), lambda i: (0, i, 0)),
            pl.BlockSpec((NB, A), lambda i: (i, 0)),
            pl.BlockSpec((O * A, O), lambda i: (0, 0)),
            pl.BlockSpec((O, O), lambda i: (0, 0)),
        ],
        out_specs=pl.BlockSpec((NB, O), lambda i: (i, 0)),
        out_shape=jax.ShapeDtypeStruct((N, O), jnp.float32),
    )(partials, node_attrs, wsk, w_lin2)


def kernel(node_attrs, node_feats, edge_attrs, edge_feats, edge_index,
           W_tpw, W_lin1, W_skip, W_lin2):
    pad = E2 - E
    sender_chunks = edge_index[0].reshape(NCR, CHUNK)
    recv_chunks = edge_index[1].reshape(NCR, CHUNK)
    eft = jnp.pad(edge_feats.T, ((0, 0), (0, pad)))
    eat = jnp.pad(edge_attrs.T, ((0, 0), (0, pad)))
    wcatt = W_tpw.reshape(R * F, S * O).T
    wlin1t = W_lin1.T
    wsk = W_skip.reshape(O * A, O)

    xst = _sc_gather(node_feats, sender_chunks)
    mjit = _tc_edge(xst, eat, eft, wcatt, wlin1t)
    partials = _sc_scatter(mjit, recv_chunks)
    out = _tc_node(partials, node_attrs, wsk, W_lin2)
    return out
